# Initial kernel scaffold; baseline (speedup 1.0000x reference)
import jax
import jax.numpy as jnp
from jax.experimental import pallas as pl


def kernel(user_embedding, item_embedding, adj_row, adj_col, adj_val, W_gc_1, b_gc_1, W_bi_1, b_bi_1):
    raise NotImplementedError("write your pallas kernel here")


# speedup vs baseline: 1.5154x; 1.5154x over previous
import functools

import jax
import jax.numpy as jnp
from jax import lax
from jax.experimental import pallas as pl
from jax.experimental.pallas import tpu as pltpu
from jax.experimental.pallas import tpu_sc as plsc

_CHUNK = 80
_QDIM = 16

_BCAST_DNUMS = lax.GatherDimensionNumbers(
    offset_dims=(), collapsed_slice_dims=(0,), start_index_map=(0,))


def _bcast_lane(v, j):
    idx = jnp.full((16, 1), j, jnp.int32)
    return lax.gather(v, idx, _BCAST_DNUMS, (1,),
                      mode=lax.GatherScatterMode.PROMISE_IN_BOUNDS)


def _make_sc_spmm(n_nodes, n_edges):
    n_subcores = 16
    edges_per_tile = n_edges // n_subcores
    n_chunks = edges_per_tile // _CHUNK
    rows_per_tile = -(-(n_nodes // n_subcores) // 8) * 8
    n_pad = rows_per_tile * n_subcores
    mesh = plsc.VectorSubcoreMesh(core_axis_name="c", subcore_axis_name="s")

    @functools.partial(
        pl.kernel,
        out_type=[jax.ShapeDtypeStruct((n_pad, _QDIM), jnp.float32)
                  for _ in range(4)],
        mesh=mesh,
        compiler_params=pltpu.CompilerParams(needs_layout_passes=False,
                                             use_tc_tiling_on_sc=False),
        scratch_types=[
            pltpu.VMEM((_CHUNK,), jnp.int32),
            pltpu.VMEM((_CHUNK,), jnp.int32),
            pltpu.VMEM((_CHUNK,), jnp.float32),
            pltpu.VMEM((_CHUNK, _QDIM), jnp.float32),
            pltpu.VMEM((rows_per_tile, _QDIM), jnp.float32),
            pltpu.VMEM_SHARED((n_pad, _QDIM), jnp.float32),
            pltpu.SemaphoreType.DMA,
        ],
    )
    def sc_spmm(ego4_hbm, row_hbm, col40_hbm, col41_hbm, col42_hbm, col43_hbm,
                val_hbm,
                out0_hbm, out1_hbm, out2_hbm, out3_hbm,
                col4_v, row_v, val_v, rows_v, zbuf_v, acc_sh, sem):
        c = lax.axis_index("c")
        s = lax.axis_index("s")

        zero16 = jnp.zeros((_QDIM,), jnp.float32)

        def zfill(i, _):
            zbuf_v[i, pl.ds(0, _QDIM)] = zero16
            return 0

        lax.fori_loop(0, rows_per_tile, zfill, 0)

        def zero_acc():
            pltpu.sync_copy(zbuf_v, acc_sh.at[pl.ds(s * rows_per_tile,
                                                    rows_per_tile)])

        def run_pass(col4q_hbm, out_hbm):
            def chunk_body(i, _):
                base = s * edges_per_tile + i * _CHUNK
                pltpu.sync_copy(col4q_hbm.at[pl.ds(base, _CHUNK)], col4_v)
                pltpu.sync_copy(row_hbm.at[pl.ds(base, _CHUNK)], row_v)
                pltpu.sync_copy(val_hbm.at[pl.ds(base, _CHUNK)], val_v)
                pltpu.async_copy(ego4_hbm.at[col4_v], rows_v, sem).wait()

                for g in range(_CHUNK // 16):
                    vv = val_v[pl.ds(g * 16, 16)]
                    for j in range(16):
                        ve = _bcast_lane(vv, j)
                        e = g * 16 + j
                        rows_v[e, pl.ds(0, _QDIM)] = (
                            rows_v[e, pl.ds(0, _QDIM)] * ve)

                pltpu.sync_copy(rows_v, acc_sh.at[row_v], add=True)
                return 0

            zero_acc()
            plsc.subcore_barrier()
            lax.fori_loop(0, n_chunks, chunk_body, 0)
            plsc.subcore_barrier()
            pltpu.sync_copy(
                acc_sh.at[pl.ds(s * rows_per_tile, rows_per_tile)],
                out_hbm.at[pl.ds(s * rows_per_tile, rows_per_tile)])
            plsc.subcore_barrier()

        @pl.when(c == 0)
        def _():
            run_pass(col40_hbm, out0_hbm)
            run_pass(col41_hbm, out1_hbm)

        @pl.when(c == 1)
        def _():
            run_pass(col42_hbm, out2_hbm)
            run_pass(col43_hbm, out3_hbm)

    return sc_spmm


def _tc_dense_body(s0_ref, s1_ref, s2_ref, s3_ref, ego_ref, wgc_ref, bgc_ref,
                   wbi_ref, bbi_ref, out_ref):
    side = jnp.concatenate(
        [s0_ref[...], s1_ref[...], s2_ref[...], s3_ref[...]], axis=1)
    ego = ego_ref[...]
    sum_e = jnp.dot(side, wgc_ref[...],
                    preferred_element_type=jnp.float32) + bgc_ref[...]
    sum_e = jnp.where(sum_e >= 0, sum_e, 0.01 * sum_e)
    bi = jnp.dot(ego * side, wbi_ref[...],
                 preferred_element_type=jnp.float32) + bbi_ref[...]
    bi = jnp.where(bi >= 0, bi, 0.01 * bi)
    eo = sum_e + bi
    norms = jnp.sqrt(jnp.sum(eo * eo, axis=1, keepdims=True))
    ne = eo / jnp.maximum(norms, 1e-12)
    out_ref[...] = jnp.concatenate([ego, ne], axis=1)


def _tc_dense(sides, ego, w_gc, b_gc, w_bi, b_bi):
    n_nodes, dim = ego.shape
    blk = 1000
    grid = (n_nodes // blk,)
    return pl.pallas_call(
        _tc_dense_body,
        grid=grid,
        in_specs=[pl.BlockSpec((blk, _QDIM), lambda i: (i, 0))
                  for _ in range(4)] + [
            pl.BlockSpec((blk, dim), lambda i: (i, 0)),
            pl.BlockSpec((dim, dim), lambda i: (0, 0)),
            pl.BlockSpec((1, dim), lambda i: (0, 0)),
            pl.BlockSpec((dim, dim), lambda i: (0, 0)),
            pl.BlockSpec((1, dim), lambda i: (0, 0)),
        ],
        out_specs=pl.BlockSpec((blk, 2 * dim), lambda i: (i, 0)),
        out_shape=jax.ShapeDtypeStruct((n_nodes, 2 * dim), jnp.float32),
    )(*sides, ego, w_gc, b_gc, w_bi, b_bi)


@jax.jit
def kernel(user_embedding, item_embedding, adj_row, adj_col, adj_val,
           W_gc_1, b_gc_1, W_bi_1, b_bi_1):
    n_users = user_embedding.shape[0]
    dim = user_embedding.shape[1]
    n_edges = adj_row.shape[0]
    ego = jnp.concatenate([user_embedding, item_embedding], axis=0)
    n_nodes = ego.shape[0]
    ego4 = ego.reshape(n_nodes * (dim // _QDIM), _QDIM)
    row = adj_row.astype(jnp.int32)
    col = adj_col.astype(jnp.int32)
    col4 = col * 4
    sc_spmm = _make_sc_spmm(n_nodes, n_edges)
    sides = sc_spmm(ego4, row, col4, col4 + 1, col4 + 2, col4 + 3, adj_val)
    all_emb = _tc_dense(sides, ego, W_gc_1, b_gc_1, W_bi_1, b_bi_1)
    return all_emb[:n_users], all_emb[n_users:]


# speedup vs baseline: 2.8484x; 1.8797x over previous
import functools

import jax
import jax.numpy as jnp
from jax import lax
from jax.experimental import pallas as pl
from jax.experimental.pallas import tpu as pltpu
from jax.experimental.pallas import tpu_sc as plsc

_CHUNK = 128
_QDIM = 16

_BCAST_DNUMS = lax.GatherDimensionNumbers(
    offset_dims=(), collapsed_slice_dims=(0,), start_index_map=(0,))


def _bcast_lane(v, j):
    idx = jnp.full((16, 1), j, jnp.int32)
    return lax.gather(v, idx, _BCAST_DNUMS, (1,),
                      mode=lax.GatherScatterMode.PROMISE_IN_BOUNDS)


def _make_sc_spmm(n_nodes, n_edges_pad):
    n_subcores = 16
    edges_per_tile = n_edges_pad // n_subcores
    n_chunks = edges_per_tile // _CHUNK
    n_pairs = n_chunks // 2
    rows_per_tile = -(-(n_nodes // n_subcores) // 8) * 8
    n_pad = rows_per_tile * n_subcores
    mesh = plsc.VectorSubcoreMesh(core_axis_name="c", subcore_axis_name="s")

    idx_t = pltpu.VMEM((_CHUNK,), jnp.int32)
    val_t = pltpu.VMEM((_CHUNK,), jnp.float32)
    rows_t = pltpu.VMEM((_CHUNK, _QDIM), jnp.float32)

    @functools.partial(
        pl.kernel,
        out_type=[jax.ShapeDtypeStruct((n_pad, _QDIM), jnp.float32)
                  for _ in range(4)],
        mesh=mesh,
        compiler_params=pltpu.CompilerParams(needs_layout_passes=False,
                                             use_tc_tiling_on_sc=False),
        scratch_types=[
            idx_t, idx_t,
            idx_t, idx_t,
            val_t, val_t,
            rows_t, rows_t,
            pltpu.VMEM((rows_per_tile, _QDIM), jnp.float32),
            pltpu.VMEM_SHARED((n_pad, _QDIM), jnp.float32),
            pltpu.SemaphoreType.DMA,
            pltpu.SemaphoreType.DMA,
        ],
    )
    def sc_spmm(ego4_hbm, row_hbm, col40_hbm, col41_hbm, col42_hbm, col43_hbm,
                val_hbm,
                out0_hbm, out1_hbm, out2_hbm, out3_hbm,
                col4_a, col4_b, row_a, row_b, val_a, val_b, rows_a, rows_b,
                zbuf_v, acc_sh, sem_a, sem_b):
        c = lax.axis_index("c")
        s = lax.axis_index("s")

        zero16 = jnp.zeros((_QDIM,), jnp.float32)

        def zfill(i, _):
            zbuf_v[i, pl.ds(0, _QDIM)] = zero16
            return 0

        lax.fori_loop(0, rows_per_tile, zfill, 0)

        def zero_acc():
            pltpu.sync_copy(zbuf_v, acc_sh.at[pl.ds(s * rows_per_tile,
                                                    rows_per_tile)])

        def run_pass(col4q_hbm, out_hbm):
            def load_idx(i, col4_v, row_v, val_v):
                base = s * edges_per_tile + i * _CHUNK
                pltpu.sync_copy(col4q_hbm.at[pl.ds(base, _CHUNK)], col4_v)
                pltpu.sync_copy(row_hbm.at[pl.ds(base, _CHUNK)], row_v)
                pltpu.sync_copy(val_hbm.at[pl.ds(base, _CHUNK)], val_v)

            def fire(col4_v, rows_v, sem):
                pltpu.async_copy(ego4_hbm.at[col4_v], rows_v, sem)

            def wait(col4_v, rows_v, sem):
                pltpu.make_async_copy(ego4_hbm.at[col4_v], rows_v, sem).wait()

            def process(row_v, val_v, rows_v):
                for g in range(_CHUNK // 16):
                    vv = val_v[pl.ds(g * 16, 16)]
                    for j in range(16):
                        ve = _bcast_lane(vv, j)
                        e = g * 16 + j
                        rows_v[e, pl.ds(0, _QDIM)] = (
                            rows_v[e, pl.ds(0, _QDIM)] * ve)
                pltpu.sync_copy(rows_v, acc_sh.at[row_v], add=True)

            zero_acc()
            plsc.subcore_barrier()

            load_idx(0, col4_a, row_a, val_a)
            fire(col4_a, rows_a, sem_a)

            def pair_body(p, _):
                load_idx(2 * p + 1, col4_b, row_b, val_b)
                fire(col4_b, rows_b, sem_b)
                wait(col4_a, rows_a, sem_a)
                process(row_a, val_a, rows_a)

                @pl.when(p < n_pairs - 1)
                def _():
                    load_idx(2 * p + 2, col4_a, row_a, val_a)
                    fire(col4_a, rows_a, sem_a)

                wait(col4_b, rows_b, sem_b)
                process(row_b, val_b, rows_b)
                return 0

            lax.fori_loop(0, n_pairs, pair_body, 0)
            plsc.subcore_barrier()
            pltpu.sync_copy(
                acc_sh.at[pl.ds(s * rows_per_tile, rows_per_tile)],
                out_hbm.at[pl.ds(s * rows_per_tile, rows_per_tile)])
            plsc.subcore_barrier()

        @pl.when(c == 0)
        def _():
            run_pass(col40_hbm, out0_hbm)
            run_pass(col41_hbm, out1_hbm)

        @pl.when(c == 1)
        def _():
            run_pass(col42_hbm, out2_hbm)
            run_pass(col43_hbm, out3_hbm)

    return sc_spmm


def _tc_dense_body(s0_ref, s1_ref, s2_ref, s3_ref, ego_ref, wgc_ref, bgc_ref,
                   wbi_ref, bbi_ref, out_ref):
    side = jnp.concatenate(
        [s0_ref[...], s1_ref[...], s2_ref[...], s3_ref[...]], axis=1)
    ego = ego_ref[...]
    sum_e = jnp.dot(side, wgc_ref[...],
                    preferred_element_type=jnp.float32) + bgc_ref[...]
    sum_e = jnp.where(sum_e >= 0, sum_e, 0.01 * sum_e)
    bi = jnp.dot(ego * side, wbi_ref[...],
                 preferred_element_type=jnp.float32) + bbi_ref[...]
    bi = jnp.where(bi >= 0, bi, 0.01 * bi)
    eo = sum_e + bi
    norms = jnp.sqrt(jnp.sum(eo * eo, axis=1, keepdims=True))
    ne = eo / jnp.maximum(norms, 1e-12)
    out_ref[...] = jnp.concatenate([ego, ne], axis=1)


def _tc_dense(sides, ego, w_gc, b_gc, w_bi, b_bi):
    n_nodes, dim = ego.shape
    blk = 1000
    grid = (n_nodes // blk,)
    return pl.pallas_call(
        _tc_dense_body,
        grid=grid,
        in_specs=[pl.BlockSpec((blk, _QDIM), lambda i: (i, 0))
                  for _ in range(4)] + [
            pl.BlockSpec((blk, dim), lambda i: (i, 0)),
            pl.BlockSpec((dim, dim), lambda i: (0, 0)),
            pl.BlockSpec((1, dim), lambda i: (0, 0)),
            pl.BlockSpec((dim, dim), lambda i: (0, 0)),
            pl.BlockSpec((1, dim), lambda i: (0, 0)),
        ],
        out_specs=pl.BlockSpec((blk, 2 * dim), lambda i: (i, 0)),
        out_shape=jax.ShapeDtypeStruct((n_nodes, 2 * dim), jnp.float32),
    )(*sides, ego, w_gc, b_gc, w_bi, b_bi)


@jax.jit
def kernel(user_embedding, item_embedding, adj_row, adj_col, adj_val,
           W_gc_1, b_gc_1, W_bi_1, b_bi_1):
    n_users = user_embedding.shape[0]
    dim = user_embedding.shape[1]
    n_edges = adj_row.shape[0]
    ego = jnp.concatenate([user_embedding, item_embedding], axis=0)
    n_nodes = ego.shape[0]
    ego4 = ego.reshape(n_nodes * (dim // _QDIM), _QDIM)
    unit = 16 * _CHUNK * 2
    n_edges_pad = -(-n_edges // unit) * unit
    pad = n_edges_pad - n_edges
    row = jnp.pad(adj_row.astype(jnp.int32), (0, pad))
    col4 = jnp.pad(adj_col.astype(jnp.int32), (0, pad)) * 4
    val = jnp.pad(adj_val, (0, pad))
    sc_spmm = _make_sc_spmm(n_nodes, n_edges_pad)
    sides = sc_spmm(ego4, row, col4, col4 + 1, col4 + 2, col4 + 3, val)
    all_emb = _tc_dense(sides, ego, W_gc_1, b_gc_1, W_bi_1, b_bi_1)
    return all_emb[:n_users], all_emb[n_users:]


# speedup vs baseline: 6.3289x; 2.2219x over previous
import functools

import jax
import jax.numpy as jnp
from jax import lax
from jax.experimental import pallas as pl
from jax.experimental.pallas import tpu as pltpu
from jax.experimental.pallas import tpu_sc as plsc

_CHUNK = 128
_QDIM = 16

_BCAST_DNUMS = lax.GatherDimensionNumbers(
    offset_dims=(), collapsed_slice_dims=(0,), start_index_map=(0,))


def _bcast_lane(v, j):
    idx = jnp.full((16, 1), j, jnp.int32)
    return lax.gather(v, idx, _BCAST_DNUMS, (1,),
                      mode=lax.GatherScatterMode.PROMISE_IN_BOUNDS)


def _make_sc_spmm(n_nodes, n_edges_pad):
    n_subcores = 16
    edges_per_tile = n_edges_pad // n_subcores
    n_chunks = edges_per_tile // _CHUNK
    n_quads = n_chunks // 4
    chunks_total = n_edges_pad // _CHUNK
    rows_per_tile = -(-(n_nodes // n_subcores) // 8) * 8
    n_pad = rows_per_tile * n_subcores
    mesh = plsc.VectorSubcoreMesh(core_axis_name="c", subcore_axis_name="s")

    slab_i = pltpu.VMEM((2, _CHUNK), jnp.int32)
    slab_f = pltpu.VMEM((2, _CHUNK), jnp.float32)
    rows_t = pltpu.VMEM((_CHUNK, _QDIM), jnp.float32)

    @functools.partial(
        pl.kernel,
        out_type=[jax.ShapeDtypeStruct((n_pad, _QDIM), jnp.float32)
                  for _ in range(4)],
        mesh=mesh,
        compiler_params=pltpu.CompilerParams(needs_layout_passes=False,
                                             use_tc_tiling_on_sc=False),
        scratch_types=[
            slab_i, slab_i,
            slab_i, slab_i,
            slab_f, slab_f,
            rows_t, rows_t,
            pltpu.VMEM((rows_per_tile, _QDIM), jnp.float32),
            pltpu.VMEM_SHARED((n_pad, _QDIM), jnp.float32),
            pltpu.SemaphoreType.DMA,
            pltpu.SemaphoreType.DMA,
            pltpu.SemaphoreType.DMA,
            pltpu.SemaphoreType.DMA,
        ],
    )
    def sc_spmm(ego4_hbm, row_hbm, col40_hbm, col41_hbm, col42_hbm, col43_hbm,
                val_hbm,
                out0_hbm, out1_hbm, out2_hbm, out3_hbm,
                col4_p, col4_q, row_p, row_q, val_p, val_q, rows_a, rows_b,
                zbuf_v, acc_sh, sem_a, sem_b, sem_ip, sem_iq):
        c = lax.axis_index("c")
        s = lax.axis_index("s")
        chunk0 = s * (n_chunks)

        zero16 = jnp.zeros((_QDIM,), jnp.float32)

        def zfill(i, _):
            zbuf_v[i, pl.ds(0, _QDIM)] = zero16
            return 0

        lax.fori_loop(0, rows_per_tile, zfill, 0)

        def zero_acc():
            pltpu.sync_copy(zbuf_v, acc_sh.at[pl.ds(s * rows_per_tile,
                                                    rows_per_tile)])

        def run_pass(col4q_hbm, out_hbm):
            def fire_idx(pair, cset, sem):
                col4_v, row_v, val_v = cset
                gc = chunk0 + 2 * pair
                pltpu.async_copy(col4q_hbm.at[pl.ds(gc, 2)], col4_v, sem)
                pltpu.async_copy(row_hbm.at[pl.ds(gc, 2)], row_v, sem)
                pltpu.async_copy(val_hbm.at[pl.ds(gc, 2)], val_v, sem)

            def wait_idx(pair, cset, sem):
                col4_v, row_v, val_v = cset
                gc = chunk0 + 2 * pair
                pltpu.make_async_copy(
                    col4q_hbm.at[pl.ds(gc, 2)], col4_v, sem).wait()
                pltpu.make_async_copy(
                    row_hbm.at[pl.ds(gc, 2)], row_v, sem).wait()
                pltpu.make_async_copy(
                    val_hbm.at[pl.ds(gc, 2)], val_v, sem).wait()

            def fire_g(col4_v, k, rows_v, sem):
                pltpu.async_copy(ego4_hbm.at[col4_v.at[k]], rows_v, sem)

            def wait_g(col4_v, k, rows_v, sem):
                pltpu.make_async_copy(
                    ego4_hbm.at[col4_v.at[k]], rows_v, sem).wait()

            def process(cset, k, rows_v):
                _, row_v, val_v = cset
                for g in range(_CHUNK // 16):
                    vv = val_v[k, pl.ds(g * 16, 16)]
                    for j in range(16):
                        ve = _bcast_lane(vv, j)
                        e = g * 16 + j
                        rows_v[e, pl.ds(0, _QDIM)] = (
                            rows_v[e, pl.ds(0, _QDIM)] * ve)
                pltpu.sync_copy(rows_v, acc_sh.at[row_v.at[k]], add=True)

            P = (col4_p, row_p, val_p)
            Q = (col4_q, row_q, val_q)

            zero_acc()
            plsc.subcore_barrier()

            fire_idx(0, P, sem_ip)
            wait_idx(0, P, sem_ip)
            fire_g(col4_p, 0, rows_a, sem_a)
            fire_idx(1, Q, sem_iq)

            def quad_body(qi, _):
                fire_g(col4_p, 1, rows_b, sem_b)
                wait_g(col4_p, 0, rows_a, sem_a)
                process(P, 0, rows_a)
                wait_idx(2 * qi + 1, Q, sem_iq)
                fire_g(col4_q, 0, rows_a, sem_a)
                wait_g(col4_p, 1, rows_b, sem_b)
                process(P, 1, rows_b)

                @pl.when(qi < n_quads - 1)
                def _():
                    fire_idx(2 * qi + 2, P, sem_ip)

                fire_g(col4_q, 1, rows_b, sem_b)
                wait_g(col4_q, 0, rows_a, sem_a)
                process(Q, 0, rows_a)

                @pl.when(qi < n_quads - 1)
                def _():
                    wait_idx(2 * qi + 2, P, sem_ip)
                    fire_g(col4_p, 0, rows_a, sem_a)

                wait_g(col4_q, 1, rows_b, sem_b)
                process(Q, 1, rows_b)

                @pl.when(qi < n_quads - 1)
                def _():
                    fire_idx(2 * qi + 3, Q, sem_iq)

                return 0

            lax.fori_loop(0, n_quads, quad_body, 0)
            plsc.subcore_barrier()
            pltpu.sync_copy(
                acc_sh.at[pl.ds(s * rows_per_tile, rows_per_tile)],
                out_hbm.at[pl.ds(s * rows_per_tile, rows_per_tile)])
            plsc.subcore_barrier()

        @pl.when(c == 0)
        def _():
            run_pass(col40_hbm, out0_hbm)
            run_pass(col41_hbm, out1_hbm)

        @pl.when(c == 1)
        def _():
            run_pass(col42_hbm, out2_hbm)
            run_pass(col43_hbm, out3_hbm)

    return sc_spmm


def _tc_dense_body(s0_ref, s1_ref, s2_ref, s3_ref, ego_ref, wgc_ref, bgc_ref,
                   wbi_ref, bbi_ref, out_ref):
    side = jnp.concatenate(
        [s0_ref[...], s1_ref[...], s2_ref[...], s3_ref[...]], axis=1)
    ego = ego_ref[...]
    sum_e = jnp.dot(side, wgc_ref[...],
                    preferred_element_type=jnp.float32) + bgc_ref[...]
    sum_e = jnp.where(sum_e >= 0, sum_e, 0.01 * sum_e)
    bi = jnp.dot(ego * side, wbi_ref[...],
                 preferred_element_type=jnp.float32) + bbi_ref[...]
    bi = jnp.where(bi >= 0, bi, 0.01 * bi)
    eo = sum_e + bi
    norms = jnp.sqrt(jnp.sum(eo * eo, axis=1, keepdims=True))
    ne = eo / jnp.maximum(norms, 1e-12)
    out_ref[...] = jnp.concatenate([ego, ne], axis=1)


def _tc_dense(sides, ego, w_gc, b_gc, w_bi, b_bi):
    n_nodes, dim = ego.shape
    blk = 1000
    grid = (n_nodes // blk,)
    return pl.pallas_call(
        _tc_dense_body,
        grid=grid,
        in_specs=[pl.BlockSpec((blk, _QDIM), lambda i: (i, 0))
                  for _ in range(4)] + [
            pl.BlockSpec((blk, dim), lambda i: (i, 0)),
            pl.BlockSpec((dim, dim), lambda i: (0, 0)),
            pl.BlockSpec((1, dim), lambda i: (0, 0)),
            pl.BlockSpec((dim, dim), lambda i: (0, 0)),
            pl.BlockSpec((1, dim), lambda i: (0, 0)),
        ],
        out_specs=pl.BlockSpec((blk, 2 * dim), lambda i: (i, 0)),
        out_shape=jax.ShapeDtypeStruct((n_nodes, 2 * dim), jnp.float32),
    )(*sides, ego, w_gc, b_gc, w_bi, b_bi)


@jax.jit
def kernel(user_embedding, item_embedding, adj_row, adj_col, adj_val,
           W_gc_1, b_gc_1, W_bi_1, b_bi_1):
    n_users = user_embedding.shape[0]
    dim = user_embedding.shape[1]
    n_edges = adj_row.shape[0]
    ego = jnp.concatenate([user_embedding, item_embedding], axis=0)
    n_nodes = ego.shape[0]
    ego4 = ego.reshape(n_nodes * (dim // _QDIM), _QDIM)
    unit = 16 * _CHUNK * 4
    n_edges_pad = -(-n_edges // unit) * unit
    pad = n_edges_pad - n_edges
    nch = n_edges_pad // _CHUNK
    row = jnp.pad(adj_row.astype(jnp.int32), (0, pad)).reshape(nch, _CHUNK)
    col4 = jnp.pad(adj_col.astype(jnp.int32), (0, pad)) * 4
    val = jnp.pad(adj_val, (0, pad)).reshape(nch, _CHUNK)
    c4 = [(col4 + q).reshape(nch, _CHUNK) for q in range(4)]
    sc_spmm = _make_sc_spmm(n_nodes, n_edges_pad)
    sides = sc_spmm(ego4, row, c4[0], c4[1], c4[2], c4[3], val)
    all_emb = _tc_dense(sides, ego, W_gc_1, b_gc_1, W_bi_1, b_bi_1)
    return all_emb[:n_users], all_emb[n_users:]


# speedup vs baseline: 6.7727x; 1.0701x over previous
import functools

import jax
import jax.numpy as jnp
from jax import lax
from jax.experimental import pallas as pl
from jax.experimental.pallas import tpu as pltpu
from jax.experimental.pallas import tpu_sc as plsc

_CHUNK = 128
_QDIM = 16

_BCAST_DNUMS = lax.GatherDimensionNumbers(
    offset_dims=(), collapsed_slice_dims=(0,), start_index_map=(0,))


def _bcast_lane(v, j):
    idx = jnp.full((16, 1), j, jnp.int32)
    return lax.gather(v, idx, _BCAST_DNUMS, (1,),
                      mode=lax.GatherScatterMode.PROMISE_IN_BOUNDS)


def _make_sc_spmm(n_nodes, n_edges_pad):
    n_subcores = 16
    edges_per_tile = n_edges_pad // n_subcores
    n_chunks = edges_per_tile // _CHUNK
    n_quads = n_chunks // 4
    chunks_total = n_edges_pad // _CHUNK
    rows_per_tile = -(-(n_nodes // n_subcores) // 8) * 8
    n_pad = rows_per_tile * n_subcores
    mesh = plsc.VectorSubcoreMesh(core_axis_name="c", subcore_axis_name="s")

    slab_i = pltpu.VMEM((2, _CHUNK), jnp.int32)
    slab_f = pltpu.VMEM((2, _CHUNK), jnp.float32)
    rows_t = pltpu.VMEM((_CHUNK, _QDIM), jnp.float32)

    @functools.partial(
        pl.kernel,
        out_type=[jax.ShapeDtypeStruct((n_pad, _QDIM), jnp.float32)
                  for _ in range(4)],
        mesh=mesh,
        compiler_params=pltpu.CompilerParams(needs_layout_passes=False,
                                             use_tc_tiling_on_sc=False),
        scratch_types=[
            slab_i, slab_i,
            slab_i, slab_i,
            slab_f, slab_f,
            rows_t, rows_t, rows_t, rows_t,
            pltpu.VMEM((_CHUNK,), jnp.int32),
            pltpu.VMEM((_CHUNK,), jnp.int32),
            pltpu.VMEM((_CHUNK,), jnp.int32),
            pltpu.VMEM((_CHUNK,), jnp.int32),
            pltpu.VMEM((rows_per_tile, _QDIM), jnp.float32),
            pltpu.VMEM_SHARED((n_pad, _QDIM), jnp.float32),
            pltpu.SemaphoreType.DMA,
            pltpu.SemaphoreType.DMA,
            pltpu.SemaphoreType.DMA,
            pltpu.SemaphoreType.DMA,
            pltpu.SemaphoreType.DMA,
            pltpu.SemaphoreType.DMA,
            pltpu.SemaphoreType.DMA,
            pltpu.SemaphoreType.DMA,
        ],
    )
    def sc_spmm(ego4_hbm, row_hbm, col40_hbm, col41_hbm, col42_hbm, col43_hbm,
                val_hbm,
                out0_hbm, out1_hbm, out2_hbm, out3_hbm,
                col4_p, col4_q, row_p, row_q, val_p, val_q,
                rows_0, rows_1, rows_2, rows_3,
                ridx_0, ridx_1, ridx_2, ridx_3,
                zbuf_v, acc_sh, sem_a, sem_b, sem_ip, sem_iq,
                ss_0, ss_1, ss_2, ss_3):
        c = lax.axis_index("c")
        s = lax.axis_index("s")
        chunk0 = s * (n_chunks)

        zero16 = jnp.zeros((_QDIM,), jnp.float32)

        def zfill(i, _):
            zbuf_v[i, pl.ds(0, _QDIM)] = zero16
            return 0

        lax.fori_loop(0, rows_per_tile, zfill, 0)

        def zero_acc():
            pltpu.sync_copy(zbuf_v, acc_sh.at[pl.ds(s * rows_per_tile,
                                                    rows_per_tile)])

        def run_pass(col4q_hbm, out_hbm):
            def fire_idx(pair, cset, sem):
                col4_v, row_v, val_v = cset
                gc = chunk0 + 2 * pair
                pltpu.async_copy(col4q_hbm.at[pl.ds(gc, 2)], col4_v, sem)
                pltpu.async_copy(row_hbm.at[pl.ds(gc, 2)], row_v, sem)
                pltpu.async_copy(val_hbm.at[pl.ds(gc, 2)], val_v, sem)

            def wait_idx(pair, cset, sem):
                col4_v, row_v, val_v = cset
                gc = chunk0 + 2 * pair
                pltpu.make_async_copy(
                    col4q_hbm.at[pl.ds(gc, 2)], col4_v, sem).wait()
                pltpu.make_async_copy(
                    row_hbm.at[pl.ds(gc, 2)], row_v, sem).wait()
                pltpu.make_async_copy(
                    val_hbm.at[pl.ds(gc, 2)], val_v, sem).wait()

            def fire_g(col4_v, k, rows_v, sem):
                pltpu.async_copy(ego4_hbm.at[col4_v.at[k]], rows_v, sem)

            def wait_g(col4_v, k, rows_v, sem):
                pltpu.make_async_copy(
                    ego4_hbm.at[col4_v.at[k]], rows_v, sem).wait()

            def process(cset, k, rows_v, ridx_v, ss):
                _, row_v, val_v = cset
                for g in range(_CHUNK // 16):
                    vv = val_v[k, pl.ds(g * 16, 16)]
                    ridx_v[pl.ds(g * 16, 16)] = row_v[k, pl.ds(g * 16, 16)]
                    for j in range(16):
                        ve = _bcast_lane(vv, j)
                        e = g * 16 + j
                        rows_v[e, pl.ds(0, _QDIM)] = (
                            rows_v[e, pl.ds(0, _QDIM)] * ve)
                pltpu.async_copy(rows_v, acc_sh.at[ridx_v], ss, add=True)

            def wait_s(rows_v, ridx_v, ss):
                pltpu.make_async_copy(
                    rows_v, acc_sh.at[ridx_v], ss).wait()

            P = (col4_p, row_p, val_p)
            Q = (col4_q, row_q, val_q)

            zero_acc()
            plsc.subcore_barrier()

            fire_idx(0, P, sem_ip)
            wait_idx(0, P, sem_ip)
            fire_g(col4_p, 0, rows_0, sem_a)
            fire_idx(1, Q, sem_iq)

            def quad_body(qi, _):
                @pl.when(qi > 0)
                def _():
                    wait_s(rows_1, ridx_1, ss_1)

                fire_g(col4_p, 1, rows_1, sem_b)
                wait_g(col4_p, 0, rows_0, sem_a)
                process(P, 0, rows_0, ridx_0, ss_0)
                wait_idx(2 * qi + 1, Q, sem_iq)

                @pl.when(qi > 0)
                def _():
                    wait_s(rows_2, ridx_2, ss_2)

                fire_g(col4_q, 0, rows_2, sem_a)
                wait_g(col4_p, 1, rows_1, sem_b)
                process(P, 1, rows_1, ridx_1, ss_1)

                @pl.when(qi < n_quads - 1)
                def _():
                    fire_idx(2 * qi + 2, P, sem_ip)

                @pl.when(qi > 0)
                def _():
                    wait_s(rows_3, ridx_3, ss_3)

                fire_g(col4_q, 1, rows_3, sem_b)
                wait_g(col4_q, 0, rows_2, sem_a)
                process(Q, 0, rows_2, ridx_2, ss_2)

                @pl.when(qi < n_quads - 1)
                def _():
                    wait_idx(2 * qi + 2, P, sem_ip)
                    wait_s(rows_0, ridx_0, ss_0)
                    fire_g(col4_p, 0, rows_0, sem_a)

                wait_g(col4_q, 1, rows_3, sem_b)
                process(Q, 1, rows_3, ridx_3, ss_3)

                @pl.when(qi < n_quads - 1)
                def _():
                    fire_idx(2 * qi + 3, Q, sem_iq)

                return 0

            lax.fori_loop(0, n_quads, quad_body, 0)
            wait_s(rows_1, ridx_1, ss_1)
            wait_s(rows_2, ridx_2, ss_2)
            wait_s(rows_3, ridx_3, ss_3)
            wait_s(rows_0, ridx_0, ss_0)
            plsc.subcore_barrier()
            pltpu.sync_copy(
                acc_sh.at[pl.ds(s * rows_per_tile, rows_per_tile)],
                out_hbm.at[pl.ds(s * rows_per_tile, rows_per_tile)])
            plsc.subcore_barrier()

        @pl.when(c == 0)
        def _():
            run_pass(col40_hbm, out0_hbm)
            run_pass(col41_hbm, out1_hbm)

        @pl.when(c == 1)
        def _():
            run_pass(col42_hbm, out2_hbm)
            run_pass(col43_hbm, out3_hbm)

    return sc_spmm


def _tc_dense_body(s0_ref, s1_ref, s2_ref, s3_ref, ego_ref, wgc_ref, bgc_ref,
                   wbi_ref, bbi_ref, out_ref):
    side = jnp.concatenate(
        [s0_ref[...], s1_ref[...], s2_ref[...], s3_ref[...]], axis=1)
    ego = ego_ref[...]
    sum_e = jnp.dot(side, wgc_ref[...],
                    preferred_element_type=jnp.float32) + bgc_ref[...]
    sum_e = jnp.where(sum_e >= 0, sum_e, 0.01 * sum_e)
    bi = jnp.dot(ego * side, wbi_ref[...],
                 preferred_element_type=jnp.float32) + bbi_ref[...]
    bi = jnp.where(bi >= 0, bi, 0.01 * bi)
    eo = sum_e + bi
    norms = jnp.sqrt(jnp.sum(eo * eo, axis=1, keepdims=True))
    ne = eo / jnp.maximum(norms, 1e-12)
    out_ref[...] = jnp.concatenate([ego, ne], axis=1)


def _tc_dense(sides, ego, w_gc, b_gc, w_bi, b_bi):
    n_nodes, dim = ego.shape
    blk = 1000
    grid = (n_nodes // blk,)
    return pl.pallas_call(
        _tc_dense_body,
        grid=grid,
        in_specs=[pl.BlockSpec((blk, _QDIM), lambda i: (i, 0))
                  for _ in range(4)] + [
            pl.BlockSpec((blk, dim), lambda i: (i, 0)),
            pl.BlockSpec((dim, dim), lambda i: (0, 0)),
            pl.BlockSpec((1, dim), lambda i: (0, 0)),
            pl.BlockSpec((dim, dim), lambda i: (0, 0)),
            pl.BlockSpec((1, dim), lambda i: (0, 0)),
        ],
        out_specs=pl.BlockSpec((blk, 2 * dim), lambda i: (i, 0)),
        out_shape=jax.ShapeDtypeStruct((n_nodes, 2 * dim), jnp.float32),
    )(*sides, ego, w_gc, b_gc, w_bi, b_bi)


@jax.jit
def kernel(user_embedding, item_embedding, adj_row, adj_col, adj_val,
           W_gc_1, b_gc_1, W_bi_1, b_bi_1):
    n_users = user_embedding.shape[0]
    dim = user_embedding.shape[1]
    n_edges = adj_row.shape[0]
    ego = jnp.concatenate([user_embedding, item_embedding], axis=0)
    n_nodes = ego.shape[0]
    ego4 = ego.reshape(n_nodes * (dim // _QDIM), _QDIM)
    unit = 16 * _CHUNK * 4
    n_edges_pad = -(-n_edges // unit) * unit
    pad = n_edges_pad - n_edges
    nch = n_edges_pad // _CHUNK
    row = jnp.pad(adj_row.astype(jnp.int32), (0, pad)).reshape(nch, _CHUNK)
    col4 = jnp.pad(adj_col.astype(jnp.int32), (0, pad)) * 4
    val = jnp.pad(adj_val, (0, pad)).reshape(nch, _CHUNK)
    c4 = [(col4 + q).reshape(nch, _CHUNK) for q in range(4)]
    sc_spmm = _make_sc_spmm(n_nodes, n_edges_pad)
    sides = sc_spmm(ego4, row, c4[0], c4[1], c4[2], c4[3], val)
    all_emb = _tc_dense(sides, ego, W_gc_1, b_gc_1, W_bi_1, b_bi_1)
    return all_emb[:n_users], all_emb[n_users:]
